# Initial kernel scaffold; baseline (speedup 1.0000x reference)
import jax
import jax.numpy as jnp
from jax.experimental import pallas as pl


def kernel(x, edge, A_value, W1, b1, W2, b2):
    raise NotImplementedError("write your pallas kernel here")


# speedup vs baseline: 8.8236x; 8.8236x over previous
import functools

import jax
import jax.numpy as jnp
from jax import lax
from jax.experimental import pallas as pl
from jax.experimental.pallas import tpu as pltpu
from jax.experimental.pallas import tpu_sc as plsc

N = 10000
E = 320000
HIDDEN = 128
NCLASS = 64
GAMMA = 0.5
LAM = 0.5

NP = 10240
NW = 32
EPW = E // NW
C = 80
NCH = EPW // C
SB = 25
NBLK = NCH // SB
G = C // 16


def _make_sc_round(F):
    Fa = F + 16
    KV = Fa // 16
    mesh = plsc.VectorSubcoreMesh(core_axis_name="c", subcore_axis_name="s")
    rows_per = NP // 16

    @functools.partial(
        pl.kernel,
        out_type=jax.ShapeDtypeStruct((2, NP, Fa), jnp.float32),
        mesh=mesh,
        compiler_params=pltpu.CompilerParams(
            needs_layout_passes=False, use_tc_tiling_on_sc=False),
        scratch_types=[
            pltpu.VMEM((SB, C), jnp.int32),
            pltpu.VMEM((SB, C), jnp.int32),
            pltpu.VMEM((SB, C), jnp.float32),
            pltpu.VMEM((C, Fa), jnp.float32),
            pltpu.VMEM((C, Fa), jnp.float32),
            pltpu.VMEM((256,), jnp.float32),
            pltpu.VMEM((16,), jnp.float32),
            pltpu.VMEM_SHARED((NP, Fa), jnp.float32),
            pltpu.SemaphoreType.DMA,
            pltpu.SemaphoreType.DMA,
        ],
    )
    def sc_round(ha_hbm, xa_hbm, src_hbm, dst_hbm, a_hbm, z_hbm, u_out,
                 srcb, dstb, ab, bufh, bufx, ps, sbuf, ush, semh, semx):
        cid = lax.axis_index("c")
        sid = lax.axis_index("s")
        wid = cid * 16 + sid

        pltpu.sync_copy(z_hbm.at[pl.ds(sid * rows_per, rows_per)],
                        ush.at[pl.ds(sid * rows_per, rows_per)])
        plsc.subcore_barrier()

        def block(b, carry):
            pltpu.sync_copy(src_hbm.at[wid, pl.ds(b * SB, SB)], srcb)
            pltpu.sync_copy(dst_hbm.at[wid, pl.ds(b * SB, SB)], dstb)
            pltpu.sync_copy(a_hbm.at[wid, pl.ds(b * SB, SB)], ab)

            def chunk(cc, carry2):
                cph = pltpu.async_copy(ha_hbm.at[srcb.at[cc]], bufh, semh)
                cpx = pltpu.async_copy(xa_hbm.at[dstb.at[cc]], bufx, semx)
                cph.wait()
                cpx.wait()
                for g in range(G):
                    a16 = ab[cc, pl.ds(g * 16, 16)]
                    for j in range(16):
                        e = g * 16 + j
                        acc = bufh[e, pl.ds(0, 16)] * bufx[e, pl.ds(0, 16)]
                        for k in range(1, KV):
                            acc = acc + bufh[e, pl.ds(16 * k, 16)] * bufx[e, pl.ds(16 * k, 16)]
                        ps[pl.ds(j * 16, 16)] = acc
                    rowbase = lax.iota(jnp.int32, 16) * 16
                    hx = plsc.load_gather(ps, [rowbase])
                    for l in range(1, 16):
                        hx = hx + plsc.load_gather(ps, [rowbase + l])
                    s = jnp.maximum(a16 + hx, 0.0)
                    sbuf[...] = s
                    for j in range(16):
                        e = g * 16 + j
                        sj = plsc.load_gather(sbuf, [jnp.full((16,), j, jnp.int32)])
                        for k in range(KV):
                            bufx[e, pl.ds(16 * k, 16)] = bufx[e, pl.ds(16 * k, 16)] * sj
                pltpu.sync_copy(bufx, ush.at[srcb.at[cc]], add=True)
                return carry2

            lax.fori_loop(0, SB, chunk, 0)
            return carry

        lax.fori_loop(0, NBLK, block, 0)
        plsc.subcore_barrier()
        pltpu.sync_copy(ush.at[pl.ds(sid * rows_per, rows_per)],
                        u_out.at[cid, pl.ds(sid * rows_per, rows_per)])

    return sc_round


_sc128 = _make_sc_round(HIDDEN)
_sc64 = _make_sc_round(NCLASS)


def _aug_x(x_mat, sum_col):
    n = x_mat.shape[0]
    return jnp.concatenate(
        [x_mat, sum_col, jnp.ones((n, 1), jnp.float32),
         jnp.zeros((n, 14), jnp.float32)], axis=1)


def _aug_h(h_mat, sum_col):
    n = h_mat.shape[0]
    return jnp.concatenate(
        [(2.0 * LAM) * h_mat, jnp.full((n, 1), -LAM, jnp.float32),
         (-LAM) * sum_col, jnp.zeros((n, 14), jnp.float32)], axis=1)


def _norm_from_u(u_ref, F):
    u = u_ref[0, :N] + u_ref[1, :N]
    deg = u[:, F + 1:F + 2]
    dinv = jnp.where(deg > 0.0, 1.0 / deg, 0.0)
    return u[:, :F] * dinv


def _mm1_body(x_ref, w_ref, b_ref, xa_ref, ha_ref):
    x_out = jnp.dot(x_ref[...], w_ref[...], preferred_element_type=jnp.float32) + b_ref[...]
    s = jnp.sum(x_out * x_out, axis=1, keepdims=True)
    xa_ref[...] = _aug_x(x_out, s)
    ha_ref[...] = _aug_h(x_out, s)


_mm1 = pl.pallas_call(
    _mm1_body,
    out_shape=[jax.ShapeDtypeStruct((N, HIDDEN + 16), jnp.float32),
               jax.ShapeDtypeStruct((N, HIDDEN + 16), jnp.float32)],
)


def _comb_body(u_ref, ha_ref, *, F):
    h = _norm_from_u(u_ref, F)
    s = jnp.sum(h * h, axis=1, keepdims=True)
    ha_ref[...] = _aug_h(h, s)


_comb128 = pl.pallas_call(
    functools.partial(_comb_body, F=HIDDEN),
    out_shape=jax.ShapeDtypeStruct((N, HIDDEN + 16), jnp.float32),
)
_comb64 = pl.pallas_call(
    functools.partial(_comb_body, F=NCLASS),
    out_shape=jax.ShapeDtypeStruct((N, NCLASS + 16), jnp.float32),
)


def _mm2_body(u_ref, xa1_ref, w_ref, b_ref, xa_ref, ha_ref):
    h = _norm_from_u(u_ref, HIDDEN)
    x1 = xa1_ref[:, :HIDDEN]
    xl = (h + GAMMA * x1) * (1.0 / (1.0 + GAMMA))
    hrelu = jnp.maximum(xl, 0.0)
    x2 = jnp.dot(hrelu, w_ref[...], preferred_element_type=jnp.float32) + b_ref[...]
    s = jnp.sum(x2 * x2, axis=1, keepdims=True)
    xa_ref[...] = _aug_x(x2, s)
    ha_ref[...] = _aug_h(x2, s)


_mm2 = pl.pallas_call(
    _mm2_body,
    out_shape=[jax.ShapeDtypeStruct((N, NCLASS + 16), jnp.float32),
               jax.ShapeDtypeStruct((N, NCLASS + 16), jnp.float32)],
)


def _final_body(u_ref, xa2_ref, out_ref):
    h = _norm_from_u(u_ref, NCLASS)
    x2 = xa2_ref[:, :NCLASS]
    o = (h + GAMMA * x2) * (1.0 / (1.0 + GAMMA))
    m = jnp.max(o, axis=1, keepdims=True)
    l = o - m
    lse = jnp.log(jnp.sum(jnp.exp(l), axis=1, keepdims=True))
    out_ref[...] = l - lse


_final = pl.pallas_call(
    _final_body,
    out_shape=jax.ShapeDtypeStruct((N, NCLASS), jnp.float32),
)


def kernel(x, edge, A_value, W1, b1, W2, b2):
    src = edge[0].reshape(NW, NCH, C)
    dst = edge[1].reshape(NW, NCH, C)
    a_r = A_value.reshape(NW, NCH, C)
    z144 = jnp.zeros((NP, HIDDEN + 16), jnp.float32)
    z80 = jnp.zeros((NP, NCLASS + 16), jnp.float32)
    b1r = b1.reshape(1, HIDDEN)
    b2r = b2.reshape(1, NCLASS)

    xa1, ha = _mm1(x, W1, b1r)
    u = _sc128(ha, xa1, src, dst, a_r, z144)
    ha = _comb128(u)
    u = _sc128(ha, xa1, src, dst, a_r, z144)
    xa2, ha2 = _mm2(u, xa1, W2, b2r)
    u2 = _sc64(ha2, xa2, src, dst, a_r, z80)
    ha2 = _comb64(u2)
    u2 = _sc64(ha2, xa2, src, dst, a_r, z80)
    return _final(u2, xa2)
